# baseline (speedup 1.0000x reference)
import functools

import jax
import jax.numpy as jnp
from jax import lax
from jax.experimental import pallas as pl
from jax.experimental.pallas import tpu as pltpu
from jax.experimental.pallas import tpu_sc as plsc

_B = 1024
_M = 26
_CLASSES = 1000
_NC, _NS = 2, 16
_NW = _NC * _NS
_P = 5
_CROWS = _P * 8
_CPM = _CLASSES // _CROWS
_TCH = _M * _CPM
_PER_W = _TCH // _NW
_EXTRA = _TCH - _PER_W * _NW

_mesh = plsc.VectorSubcoreMesh(core_axis_name="c", subcore_axis_name="s")


@functools.partial(
    pl.kernel,
    mesh=_mesh,
    out_type=jax.ShapeDtypeStruct((_M, _CLASSES, _B), jnp.float32),
    scratch_types=[
        pltpu.VMEM((2 * _B,), jnp.int32),
        pltpu.VMEM((_CROWS, _B), jnp.float32),
        pltpu.VMEM((_CROWS, _B), jnp.float32),
        pltpu.VMEM((_CROWS, _B), jnp.float32),
        [pltpu.SemaphoreType.DMA] * (6 * _P),
    ],
    compiler_params=pltpu.CompilerParams(needs_layout_passes=False),
)
def _onehot_sc(idx_hbm, noise_hbm, out_hbm, idx_v, buf0, buf1, buf2, sems):
    wid = lax.axis_index("s") * _NC + lax.axis_index("c")
    base = wid * _PER_W + jnp.minimum(wid, _EXTRA)
    cnt = _PER_W + jnp.where(wid < _EXTRA, 1, 0)

    m_lo = base // _CPM
    m_hi = jnp.minimum(m_lo + 1, _M - 1)
    pltpu.sync_copy(idx_hbm.at[m_lo, :], idx_v.at[pl.ds(0, _B)])
    pltpu.sync_copy(idx_hbm.at[m_hi, :], idx_v.at[pl.ds(_B, _B)])

    bufs = (buf0, buf1, buf2)

    def isem(b3, p):
        return sems[b3 * _P + p]

    def osem(b3, p):
        return sems[3 * _P + b3 * _P + p]

    def _noise(k, r0, nr):
        t = base + k
        m = t // _CPM
        j = t - m * _CPM
        return noise_hbm.at[m, pl.ds(j * _CROWS + r0, nr), :]

    def _out(k, r0, nr):
        t = base + k
        m = t // _CPM
        j = t - m * _CPM
        return out_hbm.at[m, pl.ds(j * _CROWS + r0, nr), :]

    def _start_in(k, b3):
        for p in range(_P):
            pltpu.make_async_copy(
                _noise(k, p * 8, 8), bufs[b3].at[pl.ds(p * 8, 8), :],
                isem(b3, p)).start()

    for k in range(2):
        _start_in(k, k)

    def group(gg, carry):
        for b3 in range(3):
            k = gg * 3 + b3
            buf = bufs[b3]
            d = (b3 + 2) % 3

            @pl.when(k < cnt)
            def _compute():
                t = base + k
                m = t // _CPM
                c_base = (t - m * _CPM) * _CROWS
                roff = (m - m_lo) * _B

                for p in range(_P):
                    pltpu.make_async_copy(
                        noise_hbm.at[0, pl.ds(0, 8), :],
                        buf.at[pl.ds(p * 8, 8), :], isem(b3, p)).wait()

                    @plsc.parallel_loop(0, _B // 16, unroll=4)
                    def _blk(blk, p=p):
                        b0 = blk * 16
                        idxv = idx_v[pl.ds(roff + b0, 16)]
                        for row in range(p * 8, p * 8 + 8):
                            v = buf[row, pl.ds(b0, 16)]
                            hot = jnp.where(idxv == c_base + row, 1.0, 0.0)
                            buf[row, pl.ds(b0, 16)] = v * 0.01 + hot

                    pltpu.make_async_copy(
                        buf.at[pl.ds(p * 8, 8), :], _out(k, p * 8, 8),
                        osem(b3, p)).start()

            @pl.when((k >= 1) & (k < cnt + 1))
            def _retire():
                for p in range(_P):
                    pltpu.make_async_copy(
                        bufs[d].at[pl.ds(p * 8, 8), :],
                        out_hbm.at[0, pl.ds(0, 8), :], osem(d, p)).wait()

            @pl.when(k + 2 < cnt)
            def _prefetch():
                _start_in(k + 2, d)
        return carry

    lax.fori_loop(0, (_PER_W + 1 + 2) // 3 + 1, group, 0)


def kernel(indices, noise):
    idx_t = jnp.transpose(indices.astype(jnp.int32))
    noise_t = jnp.transpose(noise, (1, 2, 0))
    out_t = _onehot_sc(idx_t, noise_t)
    return jnp.transpose(out_t, (2, 0, 1))


# speedup vs baseline: 1.0574x; 1.0574x over previous
import functools

import jax
import jax.numpy as jnp
from jax import lax
from jax.experimental import pallas as pl
from jax.experimental.pallas import tpu as pltpu
from jax.experimental.pallas import tpu_sc as plsc

_B = 1024
_M = 26
_CLASSES = 1000
_NC, _NS = 2, 16
_NW = _NC * _NS
_P = 5
_JPM = _CLASSES // 8
_TROWS = _M * _JPM
_PER_W = _TROWS // _NW
_EXTRA = _TROWS - _PER_W * _NW
_KMAX = (_PER_W + 1 + _P - 1) // _P

_mesh = plsc.VectorSubcoreMesh(core_axis_name="c", subcore_axis_name="s")


@functools.partial(
    pl.kernel,
    mesh=_mesh,
    out_type=jax.ShapeDtypeStruct((_M, _CLASSES, _B), jnp.float32),
    scratch_types=[
        pltpu.VMEM((2 * _B,), jnp.int32),
        pltpu.VMEM((_P * 8, _B), jnp.float32),
        pltpu.VMEM((_P * 8, _B), jnp.float32),
        pltpu.VMEM((_P * 8, _B), jnp.float32),
        [pltpu.SemaphoreType.DMA] * (6 * _P),
    ],
    compiler_params=pltpu.CompilerParams(needs_layout_passes=False),
)
def _onehot_sc(idx_hbm, noise_hbm, out_hbm, idx_v, buf0, buf1, buf2, sems):
    wid = lax.axis_index("s") * _NC + lax.axis_index("c")
    r0 = wid * _PER_W + jnp.minimum(wid, _EXTRA)
    nr = _PER_W + jnp.where(wid < _EXTRA, 1, 0)

    m_lo = r0 // _JPM
    m_hi = jnp.minimum(m_lo + 1, _M - 1)
    pltpu.sync_copy(idx_hbm.at[m_lo, :], idx_v.at[pl.ds(0, _B)])
    pltpu.sync_copy(idx_hbm.at[m_hi, :], idx_v.at[pl.ds(_B, _B)])

    bufs = (buf0, buf1, buf2)

    def isem(b3, p):
        return sems[b3 * _P + p]

    def osem(b3, p):
        return sems[3 * _P + b3 * _P + p]

    def _mj(k, p):
        g = r0 + k * _P + p
        m = g // _JPM
        j = g - m * _JPM
        return m, j

    def _start_in(k, b3):
        for p in range(_P):
            @pl.when(k * _P + p < nr)
            def _(p=p):
                m, j = _mj(k, p)
                pltpu.make_async_copy(
                    noise_hbm.at[m, pl.ds(j * 8, 8), :],
                    bufs[b3].at[pl.ds(p * 8, 8), :], isem(b3, p)).start()

    for k in range(2):
        _start_in(k, k)

    def group(gg, carry):
        for b3 in range(3):
            k = gg * 3 + b3
            buf = bufs[b3]
            d = (b3 + 2) % 3

            for p in range(_P):
                @pl.when(k * _P + p < nr)
                def _compute(p=p):
                    pltpu.make_async_copy(
                        noise_hbm.at[0, pl.ds(0, 8), :],
                        buf.at[pl.ds(p * 8, 8), :], isem(b3, p)).wait()
                    m, j = _mj(k, p)
                    c_base = j * 8
                    roff = (m - m_lo) * _B

                    @plsc.parallel_loop(0, _B // 16, unroll=2)
                    def _blk(blk):
                        b0 = blk * 16
                        idxv = idx_v[pl.ds(roff + b0, 16)]
                        for rr in range(8):
                            row = p * 8 + rr
                            v = buf[row, pl.ds(b0, 16)]
                            hot = jnp.where(idxv == c_base + rr, 1.0, 0.0)
                            buf[row, pl.ds(b0, 16)] = v * 0.01 + hot

                    pltpu.make_async_copy(
                        buf.at[pl.ds(p * 8, 8), :],
                        out_hbm.at[m, pl.ds(j * 8, 8), :], osem(b3, p)).start()

            for p in range(_P):
                @pl.when((k >= 1) & ((k - 1) * _P + p < nr))
                def _retire(p=p):
                    pltpu.make_async_copy(
                        bufs[d].at[pl.ds(p * 8, 8), :],
                        out_hbm.at[0, pl.ds(0, 8), :], osem(d, p)).wait()

            _start_in(k + 2, d)
        return carry

    lax.fori_loop(0, (_KMAX + 1 + 2) // 3 + 1, group, 0)


def kernel(indices, noise):
    idx_t = jnp.transpose(indices.astype(jnp.int32))
    noise_t = jnp.transpose(noise, (1, 2, 0))
    out_t = _onehot_sc(idx_t, noise_t)
    return jnp.transpose(out_t, (2, 0, 1))


# speedup vs baseline: 1.0788x; 1.0202x over previous
import functools

import jax
import jax.numpy as jnp
from jax import lax
from jax.experimental import pallas as pl
from jax.experimental.pallas import tpu as pltpu
from jax.experimental.pallas import tpu_sc as plsc

_B = 1024
_M = 26
_CLASSES = 1000
_NC, _NS = 2, 16
_NW = _NC * _NS
_P = 5
_CROWS = _P * 8
_CPM = _CLASSES // _CROWS
_TCH = _M * _CPM
_PER_W = _TCH // _NW
_EXTRA = _TCH - _PER_W * _NW

_mesh = plsc.VectorSubcoreMesh(core_axis_name="c", subcore_axis_name="s")


@functools.partial(
    pl.kernel,
    mesh=_mesh,
    out_type=jax.ShapeDtypeStruct((_M, _CLASSES, _B), jnp.float32),
    scratch_types=[
        pltpu.VMEM((2 * _B,), jnp.int32),
        pltpu.VMEM((_CROWS, _B), jnp.float32),
        pltpu.VMEM((_CROWS, _B), jnp.float32),
        pltpu.VMEM((_CROWS, _B), jnp.float32),
        [pltpu.SemaphoreType.DMA] * (6 * _P),
    ],
    compiler_params=pltpu.CompilerParams(needs_layout_passes=False),
)
def _onehot_sc(idx_hbm, noise_hbm, out_hbm, idx_v, buf0, buf1, buf2, sems):
    wid = lax.axis_index("s") * _NC + lax.axis_index("c")
    base = wid * _PER_W + jnp.minimum(wid, _EXTRA)
    cnt = _PER_W + jnp.where(wid < _EXTRA, 1, 0)

    m_lo = base // _CPM
    m_hi = jnp.minimum(m_lo + 1, _M - 1)
    pltpu.sync_copy(idx_hbm.at[m_lo, :], idx_v.at[pl.ds(0, _B)])
    pltpu.sync_copy(idx_hbm.at[m_hi, :], idx_v.at[pl.ds(_B, _B)])

    bufs = (buf0, buf1, buf2)

    def isem(b3, p):
        return sems[b3 * _P + p]

    def osem(b3, p):
        return sems[3 * _P + b3 * _P + p]

    def _noise(k, r0, nr):
        t = base + k
        m = t // _CPM
        j = t - m * _CPM
        return noise_hbm.at[m, pl.ds(j * _CROWS + r0, nr), :]

    def _out(k, r0, nr):
        t = base + k
        m = t // _CPM
        j = t - m * _CPM
        return out_hbm.at[m, pl.ds(j * _CROWS + r0, nr), :]

    def _start_in(k, b3):
        for p in range(_P):
            pltpu.make_async_copy(
                _noise(k, p * 8, 8), bufs[b3].at[pl.ds(p * 8, 8), :],
                isem(b3, p)).start()

    for k in range(2):
        _start_in(k, k)

    def group(gg, carry):
        for b3 in range(3):
            k = gg * 3 + b3
            buf = bufs[b3]
            d = (b3 + 2) % 3

            @pl.when(k < cnt)
            def _compute():
                t = base + k
                m = t // _CPM
                c_base = (t - m * _CPM) * _CROWS
                roff = (m - m_lo) * _B

                for p in range(_P):
                    pltpu.make_async_copy(
                        noise_hbm.at[0, pl.ds(0, 8), :],
                        buf.at[pl.ds(p * 8, 8), :], isem(b3, p)).wait()

                    @plsc.parallel_loop(0, _B // 16, unroll=2)
                    def _blk(blk, p=p):
                        b0 = blk * 16
                        idxv = idx_v[pl.ds(roff + b0, 16)]
                        for row in range(p * 8, p * 8 + 8):
                            v = buf[row, pl.ds(b0, 16)]
                            hot = jnp.where(idxv == c_base + row, 1.0, 0.0)
                            buf[row, pl.ds(b0, 16)] = v * 0.01 + hot

                    pltpu.make_async_copy(
                        buf.at[pl.ds(p * 8, 8), :], _out(k, p * 8, 8),
                        osem(b3, p)).start()

            @pl.when((k >= 1) & (k < cnt + 1))
            def _retire():
                for p in range(_P):
                    pltpu.make_async_copy(
                        bufs[d].at[pl.ds(p * 8, 8), :],
                        out_hbm.at[0, pl.ds(0, 8), :], osem(d, p)).wait()

            @pl.when(k + 2 < cnt)
            def _prefetch():
                _start_in(k + 2, d)
        return carry

    lax.fori_loop(0, (_PER_W + 1 + 2) // 3 + 1, group, 0)


def kernel(indices, noise):
    idx_t = jnp.transpose(indices.astype(jnp.int32))
    noise_t = jnp.transpose(noise, (1, 2, 0))
    out_t = _onehot_sc(idx_t, noise_t)
    return jnp.transpose(out_t, (2, 0, 1))
